# speedup vs baseline: 1.9906x; 1.9906x over previous
import jax
import jax.numpy as jnp
from jax.experimental import pallas as pl

_ROWS = 256


def _ln_kernel(x_ref, pos_ref, gamma_ref, beta_ref, out_ref):
    emb = x_ref[0] + pos_ref[...]
    mean = jnp.mean(emb, axis=-1, keepdims=True)
    cent = emb - mean
    var = jnp.mean(cent * cent, axis=-1, keepdims=True)
    normed = cent * jax.lax.rsqrt(var + 1e-5)
    out_ref[0] = normed * gamma_ref[...] + beta_ref[...]


def kernel(x, pos_table, ln_gamma, ln_beta):
    B, S, E = x.shape
    gamma2 = ln_gamma.reshape(1, E)
    beta2 = ln_beta.reshape(1, E)
    grid = (S // _ROWS, B)
    return pl.pallas_call(
        _ln_kernel,
        grid=grid,
        in_specs=[
            pl.BlockSpec((1, _ROWS, E), lambda s, b: (b, s, 0)),
            pl.BlockSpec((_ROWS, E), lambda s, b: (s, 0)),
            pl.BlockSpec((1, E), lambda s, b: (0, 0)),
            pl.BlockSpec((1, E), lambda s, b: (0, 0)),
        ],
        out_specs=pl.BlockSpec((1, _ROWS, E), lambda s, b: (b, s, 0)),
        out_shape=jax.ShapeDtypeStruct((B, S, E), x.dtype),
    )(x, pos_table, gamma2, beta2)


# speedup vs baseline: 2.5726x; 1.2924x over previous
import jax
import jax.numpy as jnp
from jax.experimental import pallas as pl

_ROWS = 512


def _ln_kernel(x_ref, pos_ref, gamma_ref, beta_ref, out_ref):
    emb = x_ref[0] + pos_ref[...]
    mean = jnp.mean(emb, axis=-1, keepdims=True)
    cent = emb - mean
    var = jnp.mean(cent * cent, axis=-1, keepdims=True)
    normed = cent * jax.lax.rsqrt(var + 1e-5)
    out_ref[0] = normed * gamma_ref[...] + beta_ref[...]


def kernel(x, pos_table, ln_gamma, ln_beta):
    B, S, E = x.shape
    gamma2 = ln_gamma.reshape(1, E)
    beta2 = ln_beta.reshape(1, E)
    grid = (S // _ROWS, B)
    return pl.pallas_call(
        _ln_kernel,
        grid=grid,
        in_specs=[
            pl.BlockSpec((1, _ROWS, E), lambda s, b: (b, s, 0)),
            pl.BlockSpec((_ROWS, E), lambda s, b: (s, 0)),
            pl.BlockSpec((1, E), lambda s, b: (0, 0)),
            pl.BlockSpec((1, E), lambda s, b: (0, 0)),
        ],
        out_specs=pl.BlockSpec((1, _ROWS, E), lambda s, b: (b, s, 0)),
        out_shape=jax.ShapeDtypeStruct((B, S, E), x.dtype),
    )(x, pos_table, gamma2, beta2)


# speedup vs baseline: 2.8644x; 1.1134x over previous
import jax
import jax.numpy as jnp
from jax.experimental import pallas as pl

_ROWS = 1024


def _ln_kernel(x_ref, pos_ref, gamma_ref, beta_ref, out_ref):
    emb = x_ref[0] + pos_ref[...]
    mean = jnp.mean(emb, axis=-1, keepdims=True)
    cent = emb - mean
    var = jnp.mean(cent * cent, axis=-1, keepdims=True)
    normed = cent * jax.lax.rsqrt(var + 1e-5)
    out_ref[0] = normed * gamma_ref[...] + beta_ref[...]


def kernel(x, pos_table, ln_gamma, ln_beta):
    B, S, E = x.shape
    gamma2 = ln_gamma.reshape(1, E)
    beta2 = ln_beta.reshape(1, E)
    grid = (S // _ROWS, B)
    return pl.pallas_call(
        _ln_kernel,
        grid=grid,
        in_specs=[
            pl.BlockSpec((1, _ROWS, E), lambda s, b: (b, s, 0)),
            pl.BlockSpec((_ROWS, E), lambda s, b: (s, 0)),
            pl.BlockSpec((1, E), lambda s, b: (0, 0)),
            pl.BlockSpec((1, E), lambda s, b: (0, 0)),
        ],
        out_specs=pl.BlockSpec((1, _ROWS, E), lambda s, b: (b, s, 0)),
        out_shape=jax.ShapeDtypeStruct((B, S, E), x.dtype),
    )(x, pos_table, gamma2, beta2)


# speedup vs baseline: 2.9829x; 1.0414x over previous
import jax
import jax.numpy as jnp
from jax.experimental import pallas as pl

_ROWS = 2048


def _ln_kernel(x_ref, pos_ref, gamma_ref, beta_ref, out_ref):
    emb = x_ref[0] + pos_ref[...]
    mean = jnp.mean(emb, axis=-1, keepdims=True)
    cent = emb - mean
    var = jnp.mean(cent * cent, axis=-1, keepdims=True)
    normed = cent * jax.lax.rsqrt(var + 1e-5)
    out_ref[0] = normed * gamma_ref[...] + beta_ref[...]


def kernel(x, pos_table, ln_gamma, ln_beta):
    B, S, E = x.shape
    gamma2 = ln_gamma.reshape(1, E)
    beta2 = ln_beta.reshape(1, E)
    grid = (S // _ROWS, B)
    return pl.pallas_call(
        _ln_kernel,
        grid=grid,
        in_specs=[
            pl.BlockSpec((1, _ROWS, E), lambda s, b: (b, s, 0)),
            pl.BlockSpec((_ROWS, E), lambda s, b: (s, 0)),
            pl.BlockSpec((1, E), lambda s, b: (0, 0)),
            pl.BlockSpec((1, E), lambda s, b: (0, 0)),
        ],
        out_specs=pl.BlockSpec((1, _ROWS, E), lambda s, b: (b, s, 0)),
        out_shape=jax.ShapeDtypeStruct((B, S, E), x.dtype),
    )(x, pos_table, gamma2, beta2)
